# baseline (speedup 1.0000x reference)
import jax
import jax.numpy as jnp
from jax import lax
from jax.experimental import pallas as pl
from jax.experimental.pallas import tpu as pltpu
from jax.experimental.pallas import tpu_sc as plsc

BATCH = 4096
SEQ = 50
EMB = 32
WPR = EMB // 2
LANES = 16
NC, NS = 2, 16
NW = NC * NS
BPW = BATCH // NW
RPC = 2
CHUNKS = BPW // RPC
IPC = RPC * SEQ
IPAD = 104
NBUF = 4


def _body(idx_hbm, table_hbm, out_hbm, idx_v, rows_v, out_v, gsems):
    cid = lax.axis_index("c")
    sid = lax.axis_index("s")
    wid = sid * NC + cid

    pltpu.sync_copy(idx_hbm.at[wid], idx_v)

    def gather(c, slot):
        pltpu.async_copy(table_hbm.at[idx_v.at[c]], rows_v.at[slot], gsems.at[slot])

    for b in range(NBUF):
        gather(b, b)

    inv = jnp.full((LANES,), 1.0 / SEQ, jnp.float32)
    lane2 = lax.iota(jnp.int32, LANES) * 2
    mask_hi = jnp.full((LANES,), -65536, jnp.int32)

    def unpack_lo(w):
        return plsc.bitcast(w << 16, jnp.float32)

    def unpack_hi(w):
        return plsc.bitcast(w & mask_hi, jnp.float32)

    def accumulate(slot, c):
        for r in range(RPC):
            base = r * SEQ
            w = rows_v[slot, base, pl.ds(0, WPR)]
            acc_e = unpack_lo(w)
            acc_o = unpack_hi(w)
            for s in range(1, SEQ):
                w = rows_v[slot, base + s, pl.ds(0, WPR)]
                acc_e = acc_e + unpack_lo(w)
                acc_o = acc_o + unpack_hi(w)
            out_base = (c * RPC + r) * EMB
            plsc.store_scatter(out_v, [out_base + lane2], acc_e * inv)
            plsc.store_scatter(out_v, [out_base + lane2 + 1], acc_o * inv)

    def step(i, carry):
        for b in range(NBUF):
            c = i * NBUF + b
            pltpu.make_async_copy(
                table_hbm.at[idx_v.at[c]], rows_v.at[b], gsems.at[b]
            ).wait()
            nxt = c + NBUF

            @pl.when(nxt < CHUNKS)
            def _():
                gather(nxt, b)

            accumulate(b, c)
        return carry

    lax.fori_loop(0, CHUNKS // NBUF, step, 0)

    pltpu.sync_copy(out_v, out_hbm.at[pl.ds(wid * BPW * EMB, BPW * EMB)])


def kernel(inputs, table):
    idx = inputs.astype(jnp.int32).reshape(NW, CHUNKS, IPC)
    idx = jnp.pad(idx, ((0, 0), (0, 0), (0, IPAD - IPC)))
    bits = jax.lax.bitcast_convert_type(table, jnp.int32)
    rnd = (bits + 0x7FFF + ((bits >> 16) & 1)) & ~0xFFFF
    even = jax.lax.shift_right_logical(rnd[:, 0::2], 16)
    table_w = rnd[:, 1::2] | even

    mesh = plsc.VectorSubcoreMesh(core_axis_name="c", subcore_axis_name="s")
    run = pl.kernel(
        _body,
        out_type=jax.ShapeDtypeStruct((BATCH * EMB,), jnp.float32),
        mesh=mesh,
        scratch_types=[
            pltpu.VMEM((CHUNKS, IPAD), jnp.int32),
            pltpu.VMEM((NBUF, IPAD, WPR), jnp.int32),
            pltpu.VMEM((BPW * EMB,), jnp.float32),
            pltpu.SemaphoreType.DMA((NBUF,)),
        ],
        compiler_params=pltpu.CompilerParams(
            use_tc_tiling_on_sc=False, needs_layout_passes=False
        ),
    )
    return run(idx, table_w).reshape(BATCH, EMB)


# speedup vs baseline: 12.2810x; 12.2810x over previous
import jax
import jax.numpy as jnp
from jax import lax
from jax.experimental import pallas as pl
from jax.experimental.pallas import tpu as pltpu
from jax.experimental.pallas import tpu_sc as plsc

BATCH = 4096
SEQ = 50
EMB = 32
WPR = EMB // 2
LANES = 16
NC, NS = 2, 16
NW = NC * NS
BPW = BATCH // NW
RPC = 2
CHUNKS = BPW // RPC
IPC = RPC * SEQ
IPAD = 104
NBUF = 4


def _body(idx_hbm, table_hbm, out_hbm, idx_v, rows_v, out_v, gsems):
    cid = lax.axis_index("c")
    sid = lax.axis_index("s")
    wid = sid * NC + cid

    pltpu.sync_copy(idx_hbm.at[wid], idx_v)

    def gather(c, slot):
        pltpu.async_copy(table_hbm.at[idx_v.at[c]], rows_v.at[slot], gsems.at[slot])

    for b in range(NBUF):
        gather(b, b)

    inv = jnp.full((LANES,), 1.0 / SEQ, jnp.float32)
    lane2 = lax.iota(jnp.int32, LANES) * 2

    def accumulate(slot, c):
        for r in range(RPC):
            base = r * SEQ
            a0, b0 = plsc.unpack(rows_v[slot, base, pl.ds(0, EMB)], format=plsc.PackFormat.INTERLEAVED)
            acc_a, acc_b = a0, b0
            for s in range(1, SEQ):
                a, b = plsc.unpack(rows_v[slot, base + s, pl.ds(0, EMB)], format=plsc.PackFormat.INTERLEAVED)
                acc_a = acc_a + a
                acc_b = acc_b + b
            out_base = (c * RPC + r) * EMB
            plsc.store_scatter(out_v, [out_base + lane2], acc_a * inv)
            plsc.store_scatter(out_v, [out_base + lane2 + 1], acc_b * inv)

    def step(i, carry):
        for b in range(NBUF):
            c = i * NBUF + b
            pltpu.make_async_copy(
                table_hbm.at[idx_v.at[c]], rows_v.at[b], gsems.at[b]
            ).wait()
            nxt = c + NBUF

            @pl.when(nxt < CHUNKS)
            def _():
                gather(nxt, b)

            accumulate(b, c)
        return carry

    lax.fori_loop(0, CHUNKS // NBUF, step, 0)

    pltpu.sync_copy(out_v, out_hbm.at[pl.ds(wid * BPW * EMB, BPW * EMB)])


def kernel(inputs, table):
    idx = inputs.astype(jnp.int32).reshape(NW, CHUNKS, IPC)
    idx = jnp.pad(idx, ((0, 0), (0, 0), (0, IPAD - IPC)))
    table_w = table.astype(jnp.bfloat16)

    mesh = plsc.VectorSubcoreMesh(core_axis_name="c", subcore_axis_name="s")
    run = pl.kernel(
        _body,
        out_type=jax.ShapeDtypeStruct((BATCH * EMB,), jnp.float32),
        mesh=mesh,
        scratch_types=[
            pltpu.VMEM((CHUNKS, IPAD), jnp.int32),
            pltpu.VMEM((NBUF, IPAD, EMB), jnp.bfloat16),
            pltpu.VMEM((BPW * EMB,), jnp.float32),
            pltpu.SemaphoreType.DMA((NBUF,)),
        ],
        compiler_params=pltpu.CompilerParams(
            use_tc_tiling_on_sc=False, needs_layout_passes=False
        ),
    )
    return run(idx, table_w).reshape(BATCH, EMB)


# speedup vs baseline: 13.3100x; 1.0838x over previous
import jax
import jax.numpy as jnp
from jax import lax
from jax.experimental import pallas as pl
from jax.experimental.pallas import tpu as pltpu
from jax.experimental.pallas import tpu_sc as plsc

BATCH = 4096
SEQ = 50
EMB = 32
LANES = 16
NC, NS = 2, 16
NW = NC * NS
BPW = BATCH // NW
RPC = 2
CHUNKS = BPW // RPC
IPC = RPC * SEQ
IPAD = 104
TRASH = BPW
ACC_ROWS = BPW + 8
NBUF = 8


def _body(idx_hbm, seg_hbm, table_hbm, out_hbm,
          idx_v, seg_v, rows_v, zero_v, acc_sh, gsems, ssems):
    cid = lax.axis_index("c")
    sid = lax.axis_index("s")
    wid = sid * NC + cid

    pltpu.sync_copy(idx_hbm.at[wid], idx_v)
    pltpu.sync_copy(seg_hbm, seg_v)

    def gather(c, slot):
        pltpu.async_copy(table_hbm.at[idx_v.at[c]], rows_v.at[slot], gsems.at[slot])

    for b in range(NBUF):
        gather(b, b)

    zero = jnp.zeros((LANES,), jnp.float32)
    for r in range(ACC_ROWS):
        zero_v[r, pl.ds(0, LANES)] = zero
        zero_v[r, pl.ds(LANES, LANES)] = zero
    pltpu.sync_copy(zero_v, acc_sh.at[sid])

    def step(i, carry):
        for b in range(NBUF):
            c = i * NBUF + b
            pltpu.make_async_copy(
                table_hbm.at[idx_v.at[c]], rows_v.at[b], gsems.at[b]
            ).wait()
            pltpu.async_copy(
                rows_v.at[b], acc_sh.at[sid].at[seg_v.at[c]], ssems.at[b],
                add=True,
            )
            pltpu.make_async_copy(
                rows_v.at[b], acc_sh.at[sid].at[seg_v.at[c]], ssems.at[b]
            ).wait()
            nxt = c + NBUF

            @pl.when(nxt < CHUNKS)
            def _():
                gather(nxt, b)

        return carry

    lax.fori_loop(0, CHUNKS // NBUF, step, 0)

    pltpu.sync_copy(acc_sh.at[sid], zero_v)
    inv = jnp.full((LANES,), 1.0 / SEQ, jnp.float32)
    for r in range(BPW):
        zero_v[r, pl.ds(0, LANES)] = zero_v[r, pl.ds(0, LANES)] * inv
        zero_v[r, pl.ds(LANES, LANES)] = zero_v[r, pl.ds(LANES, LANES)] * inv
    pltpu.sync_copy(zero_v.at[pl.ds(0, BPW)], out_hbm.at[pl.ds(wid * BPW, BPW)])


def kernel(inputs, table):
    idx = inputs.astype(jnp.int32).reshape(NW, CHUNKS, IPC)
    idx = jnp.pad(idx, ((0, 0), (0, 0), (0, IPAD - IPC)))
    j = jnp.arange(IPAD, dtype=jnp.int32)
    base = jnp.arange(CHUNKS, dtype=jnp.int32)[:, None] * RPC
    seg = jnp.where(j[None, :] < IPC, base + j[None, :] // SEQ, TRASH)

    mesh = plsc.VectorSubcoreMesh(core_axis_name="c", subcore_axis_name="s")
    run = pl.kernel(
        _body,
        out_type=jax.ShapeDtypeStruct((BATCH, EMB), jnp.float32),
        mesh=mesh,
        scratch_types=[
            pltpu.VMEM((CHUNKS, IPAD), jnp.int32),
            pltpu.VMEM((CHUNKS, IPAD), jnp.int32),
            pltpu.VMEM((NBUF, IPAD, EMB), jnp.float32),
            pltpu.VMEM((ACC_ROWS, EMB), jnp.float32),
            pltpu.VMEM_SHARED((NS, ACC_ROWS, EMB), jnp.float32),
            pltpu.SemaphoreType.DMA((NBUF,)),
            pltpu.SemaphoreType.DMA((NBUF,)),
        ],
        compiler_params=pltpu.CompilerParams(use_tc_tiling_on_sc=False),
    )
    return run(idx, seg, table)
